# baseline (speedup 1.0000x reference)
import functools

import jax
import jax.numpy as jnp
from jax import lax
from jax.experimental import pallas as pl
from jax.experimental.pallas import tpu as pltpu
from jax.experimental.pallas import tpu_sc as plsc

B, F, V, D = 16384, 26, 100000, 16
D2 = 8
H1, H2, OUT = 512, 256, 1
BF = B * F
NC, NS = 2, 16
NW = NC * NS
BW = B // NW
BWC = 256
NCH = BW // BWC


SPLITS = (13, 13)


def _sc_gather(idxT, tab_flat, fh, foff):
    mesh = plsc.VectorSubcoreMesh(
        core_axis_name="c", subcore_axis_name="s",
        num_cores=NC, num_subcores=NS)

    @functools.partial(
        pl.kernel,
        out_type=jax.ShapeDtypeStruct((B, fh * D), jnp.float32),
        mesh=mesh,
        scratch_types=[
            pltpu.VMEM((3, BWC), jnp.int32),
            pltpu.VMEM((3, D2, BWC), jnp.int32),
            pltpu.VMEM((3, D2, BWC, 2), jnp.float32),
            pltpu.VMEM((BWC, fh * D), jnp.float32),
            pltpu.SemaphoreType.DMA,
        ],
        compiler_params=pltpu.CompilerParams(use_tc_tiling_on_sc=False,
                                             needs_layout_passes=False),
    )
    def gather_kernel(idx_hbm, tab_hbm, out_hbm, idx_v, fidx_v, rows_v, x_v,
                      sem):
        wid = lax.axis_index("s") * NC + lax.axis_index("c")
        lanes = lax.iota(jnp.int32, 16)

        def load_and_fire(f, sl0, wb):
            pltpu.sync_copy(idx_hbm.at[f, pl.ds(wb, BWC)], idx_v.at[sl0])

            def fire(d, carry3):
                base = ((foff + f) * D2 + d) * V
                for i in range(BWC // 16):
                    sl = pl.ds(i * 16, 16)
                    fidx_v[sl0, d, sl] = idx_v[sl0, sl] + base
                pltpu.async_copy(tab_hbm.at[fidx_v.at[sl0, d]],
                                 rows_v.at[sl0, d], sem)
                return carry3

            lax.fori_loop(0, D2, fire, 0)

        def drain_and_scatter(f, sl0):
            def drain(d, carry3):
                pltpu.make_async_copy(tab_hbm.at[fidx_v.at[sl0, d]],
                                      rows_v.at[sl0, d], sem).wait()
                return carry3

            lax.fori_loop(0, D2, drain, 0)

            def scatter(d, carry3):
                for p in range(2):
                    col = jnp.full((16,), f * D + p, jnp.int32) + d * 2
                    for i in range(BWC // 16):
                        vals = plsc.load_gather(
                            rows_v.at[sl0, d],
                            [i * 16 + lanes, jnp.full((16,), p, jnp.int32)])
                        plsc.store_scatter(x_v, [i * 16 + lanes, col], vals)
                return carry3

            lax.fori_loop(0, D2, scatter, 0)

        def chunk_body(ci, carry):
            wb = wid * BW + ci * BWC
            load_and_fire(0, 0, wb)
            if fh > 1:
                load_and_fire(1, 1, wb)

            def f_body(f, carry2):
                load_and_fire(f + 2, (f + 2) % 3, wb)
                drain_and_scatter(f, f % 3)
                return carry2

            lax.fori_loop(0, max(fh - 2, 0), f_body, 0)
            if fh > 1:
                drain_and_scatter(fh - 2, (fh - 2) % 3)
            drain_and_scatter(fh - 1, (fh - 1) % 3)
            pltpu.sync_copy(x_v, out_hbm.at[pl.ds(wb, BWC)])
            return carry

        lax.fori_loop(0, NCH, chunk_body, 0)

    return gather_kernel(idxT, tab_flat)


TB = 1024


def _mlp_body(*refs):
    n = len(SPLITS)
    xrefs, w1refs = refs[:n], refs[n:2 * n]
    b1_ref, w2_ref, b2_ref, wo_ref, bo_ref, out_ref = refs[2 * n:]
    h = sum(jnp.dot(x[...], w[...], preferred_element_type=jnp.float32)
            for x, w in zip(xrefs, w1refs))
    h = jnp.maximum(h + b1_ref[...], 0.0)
    h = jnp.dot(h, w2_ref[...], preferred_element_type=jnp.float32)
    h = jnp.maximum(h + b2_ref[...], 0.0)
    out_ref[...] = (
        jnp.dot(h, wo_ref[...], preferred_element_type=jnp.float32) + bo_ref[...])


def _mlp(xparts, W1parts, b1, W2, b2, Wout, bout):
    xspecs = [pl.BlockSpec((TB, fh * D), lambda i: (i, 0)) for fh in SPLITS]
    wspecs = [pl.BlockSpec((fh * D, H1), lambda i: (0, 0)) for fh in SPLITS]
    return pl.pallas_call(
        _mlp_body,
        grid=(B // TB,),
        in_specs=xspecs + wspecs + [
            pl.BlockSpec((1, H1), lambda i: (0, 0)),
            pl.BlockSpec((H1, H2), lambda i: (0, 0)),
            pl.BlockSpec((1, H2), lambda i: (0, 0)),
            pl.BlockSpec((H2, OUT), lambda i: (0, 0)),
            pl.BlockSpec((1, OUT), lambda i: (0, 0)),
        ],
        out_specs=pl.BlockSpec((TB, OUT), lambda i: (i, 0)),
        out_shape=jax.ShapeDtypeStruct((B, OUT), jnp.float32),
    )(*xparts, *W1parts, b1, W2, b2, Wout, bout)


def kernel(indices, tables, W1, b1, W2, b2, Wout, bout):
    idxT = indices.astype(jnp.int32).T
    tab_flat = (jnp.swapaxes(tables, 1, 2).reshape(F, D2, 2, V)
                .transpose(0, 1, 3, 2).reshape(F * D2 * V, 2))
    xs, w1s, f0 = [], [], 0
    for fh in SPLITS:
        xs.append(_sc_gather(idxT[f0:f0 + fh], tab_flat, fh, f0))
        w1s.append(W1[f0 * D:(f0 + fh) * D])
        f0 += fh
    return _mlp(xs, w1s, b1.reshape(1, H1), W2, b2.reshape(1, H2),
                Wout, bout.reshape(1, OUT))


# speedup vs baseline: 55.3297x; 55.3297x over previous
import functools

import jax
import jax.numpy as jnp
from jax import lax
from jax.experimental import pallas as pl
from jax.experimental.pallas import tpu as pltpu
from jax.experimental.pallas import tpu_sc as plsc

B, F, V, D = 16384, 26, 100000, 16
H1, H2, OUT = 512, 256, 1
BF = B * F
NC, NS = 2, 16
NW = NC * NS
BW = B // NW
BWC = 256
NCH = BW // BWC


SPLITS = (13, 13)


def _sc_gather(idxT, tab_flat, fh, foff):
    mesh = plsc.VectorSubcoreMesh(
        core_axis_name="c", subcore_axis_name="s",
        num_cores=NC, num_subcores=NS)

    @functools.partial(
        pl.kernel,
        out_type=jax.ShapeDtypeStruct((B, fh * D), jnp.float32),
        mesh=mesh,
        scratch_types=[
            pltpu.VMEM((3, BWC), jnp.int32),
            pltpu.VMEM((3, D, BWC), jnp.int32),
            pltpu.VMEM((3, D, BWC), jnp.float32),
            pltpu.VMEM((BWC, fh * D), jnp.float32),
            pltpu.SemaphoreType.DMA,
        ],
        compiler_params=pltpu.CompilerParams(use_tc_tiling_on_sc=False,
                                             needs_layout_passes=False),
    )
    def gather_kernel(idx_hbm, tab_hbm, out_hbm, idx_v, fidx_v, rows_v, x_v,
                      sem):
        wid = lax.axis_index("s") * NC + lax.axis_index("c")
        lanes = lax.iota(jnp.int32, 16)

        def load_and_fire(f, sl0, wb):
            pltpu.sync_copy(idx_hbm.at[f, pl.ds(wb, BWC)], idx_v.at[sl0])

            def fire(d, carry3):
                base = (foff + f) * (D * V) + d * V
                for i in range(BWC // 16):
                    sl = pl.ds(i * 16, 16)
                    fidx_v[sl0, d, sl] = idx_v[sl0, sl] + base
                pltpu.async_copy(tab_hbm.at[fidx_v.at[sl0, d]],
                                 rows_v.at[sl0, d], sem)
                return carry3

            lax.fori_loop(0, D, fire, 0)

        def drain_and_scatter(f, sl0):
            def drain(d, carry3):
                pltpu.make_async_copy(tab_hbm.at[fidx_v.at[sl0, d]],
                                      rows_v.at[sl0, d], sem).wait()
                return carry3

            lax.fori_loop(0, D, drain, 0)

            def scatter(d, carry3):
                col = jnp.full((16,), f * D, jnp.int32) + d
                for i in range(BWC // 16):
                    vals = rows_v[sl0, d, pl.ds(i * 16, 16)]
                    plsc.store_scatter(x_v, [i * 16 + lanes, col], vals)
                return carry3

            lax.fori_loop(0, D, scatter, 0)

        def chunk_body(ci, carry):
            wb = wid * BW + ci * BWC
            load_and_fire(0, 0, wb)
            if fh > 1:
                load_and_fire(1, 1, wb)

            def f_body(f, carry2):
                load_and_fire(f + 2, (f + 2) % 3, wb)
                drain_and_scatter(f, f % 3)
                return carry2

            lax.fori_loop(0, max(fh - 2, 0), f_body, 0)
            if fh > 1:
                drain_and_scatter(fh - 2, (fh - 2) % 3)
            drain_and_scatter(fh - 1, (fh - 1) % 3)
            pltpu.sync_copy(x_v, out_hbm.at[pl.ds(wb, BWC)])
            return carry

        lax.fori_loop(0, NCH, chunk_body, 0)

    return gather_kernel(idxT, tab_flat)


TB = 1024


def _mlp_body(*refs):
    n = len(SPLITS)
    xrefs, w1refs = refs[:n], refs[n:2 * n]
    b1_ref, w2_ref, b2_ref, wo_ref, bo_ref, out_ref = refs[2 * n:]
    h = sum(jnp.dot(x[...], w[...], preferred_element_type=jnp.float32)
            for x, w in zip(xrefs, w1refs))
    h = jnp.maximum(h + b1_ref[...], 0.0)
    h = jnp.dot(h, w2_ref[...], preferred_element_type=jnp.float32)
    h = jnp.maximum(h + b2_ref[...], 0.0)
    out_ref[...] = (
        jnp.dot(h, wo_ref[...], preferred_element_type=jnp.float32) + bo_ref[...])


def _mlp(xparts, W1parts, b1, W2, b2, Wout, bout):
    xspecs = [pl.BlockSpec((TB, fh * D), lambda i: (i, 0)) for fh in SPLITS]
    wspecs = [pl.BlockSpec((fh * D, H1), lambda i: (0, 0)) for fh in SPLITS]
    return pl.pallas_call(
        _mlp_body,
        grid=(B // TB,),
        in_specs=xspecs + wspecs + [
            pl.BlockSpec((1, H1), lambda i: (0, 0)),
            pl.BlockSpec((H1, H2), lambda i: (0, 0)),
            pl.BlockSpec((1, H2), lambda i: (0, 0)),
            pl.BlockSpec((H2, OUT), lambda i: (0, 0)),
            pl.BlockSpec((1, OUT), lambda i: (0, 0)),
        ],
        out_specs=pl.BlockSpec((TB, OUT), lambda i: (i, 0)),
        out_shape=jax.ShapeDtypeStruct((B, OUT), jnp.float32),
    )(*xparts, *W1parts, b1, W2, b2, Wout, bout)


def kernel(indices, tables, W1, b1, W2, b2, Wout, bout):
    idxT = indices.astype(jnp.int32).T
    tab_flat = jnp.swapaxes(tables, 1, 2).reshape(F * D * V)
    xs, w1s, f0 = [], [], 0
    for fh in SPLITS:
        xs.append(_sc_gather(idxT[f0:f0 + fh], tab_flat, fh, f0))
        w1s.append(W1[f0 * D:(f0 + fh) * D])
        f0 += fh
    return _mlp(xs, w1s, b1.reshape(1, H1), W2, b2.reshape(1, H2),
                Wout, bout.reshape(1, OUT))
